# Initial kernel scaffold; baseline (speedup 1.0000x reference)
import jax
import jax.numpy as jnp
from jax.experimental import pallas as pl


def kernel(x, tables):
    raise NotImplementedError("write your pallas kernel here")


# speedup vs baseline: 1.0335x; 1.0335x over previous
import functools

import jax
import jax.numpy as jnp
from jax import lax
from jax.experimental import pallas as pl
from jax.experimental.pallas import tpu as pltpu
from jax.experimental.pallas import tpu_sc as plsc

NC = 2
NS = 16
NW = NC * NS

CHUNK = 128
GC = 4
GROUP = CHUNK * GC


def _build(F, V, D, B):
    total = B * F
    per_w = total // NW
    n_groups = per_w // GROUP
    assert total % NW == 0 and per_w % GROUP == 0 and n_groups % 2 == 0

    mesh = plsc.VectorSubcoreMesh(core_axis_name="c", subcore_axis_name="s")

    @functools.partial(
        pl.kernel,
        out_type=jax.ShapeDtypeStruct((total, D), jnp.float32),
        mesh=mesh,
        compiler_params=pltpu.CompilerParams(use_tc_tiling_on_sc=False),
        scratch_types=[
            pltpu.VMEM((per_w,), jnp.int32),
            pltpu.VMEM((2, GC, CHUNK), jnp.int32),
            pltpu.VMEM((2, GROUP, D), jnp.float32),
            pltpu.SemaphoreType.DMA,
            pltpu.SemaphoreType.DMA,
            pltpu.SemaphoreType.DMA,
            pltpu.SemaphoreType.DMA,
        ],
    )
    def emb(x_hbm, tab_hbm, out_hbm, raw_v, idx_v, rows_v, gs0, gs1, os0, os1):
        wid = lax.axis_index("s") * NC + lax.axis_index("c")
        base = wid * per_w
        gsems = (gs0, gs1)
        osems = (os0, os1)

        pltpu.sync_copy(x_hbm.at[pl.ds(base, per_w)], raw_v)

        lane = lax.iota(jnp.int32, 16)

        def compute_idx(g, h):
            for j in range(GC):
                for p in range(CHUNK // 16):
                    pos = g * GROUP + j * CHUNK + p * 16
                    fld = (lane + pos) % F
                    idx_v[h, j, pl.ds(p * 16, 16)] = (
                        raw_v[pl.ds(pos, 16)] + fld * V
                    )

        def fire_and_drain(h):
            handles = [
                pltpu.async_copy(
                    tab_hbm.at[idx_v.at[h, j]],
                    rows_v.at[h, pl.ds(j * CHUNK, CHUNK)],
                    gsems[h],
                )
                for j in range(GC)
            ]
            for hd in handles:
                hd.wait()

        def issue_ow(g, h):
            return pltpu.async_copy(
                rows_v.at[h],
                out_hbm.at[pl.ds(base + g * GROUP, GROUP)],
                osems[h],
            )

        def wait_ow(h):
            pltpu.make_async_copy(
                rows_v.at[h],
                out_hbm.at[pl.ds(base, GROUP)],
                osems[h],
            ).wait()

        for h in range(2):
            compute_idx(h, h)
            fire_and_drain(h)
            issue_ow(h, h)

        def body(sg, _):
            for h in range(2):
                g = 2 * sg + h
                wait_ow(h)
                compute_idx(g, h)
                fire_and_drain(h)
                issue_ow(g, h)
            return 0

        lax.fori_loop(1, n_groups // 2, body, 0)
        wait_ow(0)
        wait_ow(1)

    return emb


def kernel(x, tables):
    F, V, D = tables.shape
    B = x.shape[0]
    xf = x.astype(jnp.int32).reshape(B * F)
    tf = tables.reshape(F * V, D)
    out = _build(F, V, D, B)(xf, tf)
    return out.reshape(B, F, D)


# speedup vs baseline: 1.2254x; 1.1856x over previous
import functools

import jax
import jax.numpy as jnp
from jax import lax
from jax.experimental import pallas as pl
from jax.experimental.pallas import tpu as pltpu
from jax.experimental.pallas import tpu_sc as plsc

NC = 2
NS = 16
NW = NC * NS

GB = 8
PACK_BLK = 2600


def _pack(F, V, D):
    half = F * V // 2
    n_blk = half // PACK_BLK

    def body(a_ref, b_ref, o_ref):
        o_ref[...] = jnp.concatenate([a_ref[...], b_ref[...]], axis=-1)

    return pl.pallas_call(
        body,
        grid=(n_blk,),
        in_specs=[
            pl.BlockSpec((PACK_BLK, D), lambda i: (i, 0)),
            pl.BlockSpec((PACK_BLK, D), lambda i: (i + n_blk, 0)),
        ],
        out_specs=pl.BlockSpec((PACK_BLK, 2 * D), lambda i: (i, 0)),
        out_shape=jax.ShapeDtypeStruct((half, 2 * D), jnp.float32),
    )


def _gather(F, V, D, B):
    total = B * F
    per_w = total // NW
    nb = per_w // F
    group = GB * F
    n_groups = nb // GB
    half_rows = F // 2 * V
    fh = F // 2
    assert total % NW == 0 and nb % GB == 0 and n_groups % 2 == 0
    assert group % 16 == 0

    mesh = plsc.VectorSubcoreMesh(core_axis_name="c", subcore_axis_name="s")

    @functools.partial(
        pl.kernel,
        out_type=jax.ShapeDtypeStruct((B, F, D), jnp.float32),
        mesh=mesh,
        scratch_types=[
            pltpu.VMEM((per_w,), jnp.int32),
            pltpu.VMEM((2, 2, 128), jnp.int32),
            pltpu.VMEM((2, GB * F, 2 * D), jnp.float32),
            pltpu.VMEM((2, GB * F, D), jnp.float32),
            pltpu.SemaphoreType.DMA,
            pltpu.SemaphoreType.DMA,
            pltpu.SemaphoreType.DMA,
            pltpu.SemaphoreType.DMA,
            pltpu.SemaphoreType.DMA,
        ],
    )
    def emb(x_hbm, pk_hbm, out_hbm, raw_v, idx_v, rows_v, out_v,
            gs0, gs1, cs, os0, os1):
        wid = lax.axis_index("s") * NC + lax.axis_index("c")
        base = wid * per_w
        b0 = wid * nb
        gsems = (gs0, gs1)
        osems = (os0, os1)

        pltpu.sync_copy(x_hbm.at[pl.ds(base, per_w)], raw_v)

        lane = lax.iota(jnp.int32, 16)

        def compute_and_fire(g, h):
            for p in range(group // 16):
                pos = g * group + p * 16
                fld = (lane + pos) % F
                r = raw_v[pl.ds(pos, 16)] + fld * V
                idx = jnp.where(fld >= fh, r - half_rows, r)
                q = p * 16
                idx_v[h, q // 128, pl.ds(q % 128, 16)] = idx
            pltpu.async_copy(
                pk_hbm.at[idx_v.at[h, 0]],
                rows_v.at[h, pl.ds(0, 128)],
                gsems[h],
            )
            pltpu.async_copy(
                pk_hbm.at[idx_v.at[h, 1, pl.ds(0, group - 128)]],
                rows_v.at[h, pl.ds(128, group - 128)],
                gsems[h],
            )

        def drain_gather(h):
            pltpu.make_async_copy(
                pk_hbm.at[idx_v.at[h, 0]],
                rows_v.at[h, pl.ds(0, 128)],
                gsems[h],
            ).wait()
            pltpu.make_async_copy(
                pk_hbm.at[idx_v.at[h, 1, pl.ds(0, group - 128)]],
                rows_v.at[h, pl.ds(128, group - 128)],
                gsems[h],
            ).wait()

        def compact_and_emit(g, h):
            def ext(j, _):
                for f in range(F):
                    off = 0 if f < fh else D
                    row = j * F + f
                    for m in range(D // 16):
                        out_v[h, row, pl.ds(m * 16, 16)] = (
                            rows_v[h, row, pl.ds(off + m * 16, 16)]
                        )
                return 0

            lax.fori_loop(0, GB, ext, 0)
            pltpu.async_copy(
                out_v.at[h].reshape(GB, F, D),
                out_hbm.at[pl.ds(b0 + g * GB, GB), :, :],
                osems[h],
            )

        def wait_ow(h):
            pltpu.make_async_copy(
                out_v.at[h].reshape(GB, F, D),
                out_hbm.at[pl.ds(b0, GB), :, :],
                osems[h],
            ).wait()

        compute_and_fire(0, 0)

        def body(sg, _):
            for h in range(2):
                g = 2 * sg + h
                nh = 1 - h

                @pl.when(g < n_groups - 1)
                def _():
                    @pl.when(g >= 1)
                    def _():
                        wait_ow(nh)
                    compute_and_fire(g + 1, nh)

                drain_gather(h)
                compact_and_emit(g, h)
            return 0

        lax.fori_loop(0, n_groups // 2, body, 0)
        wait_ow(0)
        wait_ow(1)

    return emb


def kernel(x, tables):
    F, V, D = tables.shape
    B = x.shape[0]
    xf = x.astype(jnp.int32).reshape(B * F)
    tf = tables.reshape(F * V, D)
    packed = _pack(F, V, D)(tf, tf)
    return _gather(F, V, D, B)(xf, packed)


# speedup vs baseline: 1.2260x; 1.0005x over previous
import functools

import jax
import jax.numpy as jnp
from jax import lax
from jax.experimental import pallas as pl
from jax.experimental.pallas import tpu as pltpu
from jax.experimental.pallas import tpu_sc as plsc

NC = 2
NS = 16
NW = NC * NS

GB = 8
PACK_BLK = 2600


def _pack(F, V, D):
    half = F * V // 2
    n_blk = half // PACK_BLK

    def body(a_ref, b_ref, o_ref):
        o_ref[...] = jnp.concatenate([a_ref[...], b_ref[...]], axis=-1)

    return pl.pallas_call(
        body,
        grid=(n_blk,),
        in_specs=[
            pl.BlockSpec((PACK_BLK, D), lambda i: (i, 0)),
            pl.BlockSpec((PACK_BLK, D), lambda i: (i + n_blk, 0)),
        ],
        out_specs=pl.BlockSpec((PACK_BLK, 2 * D), lambda i: (i, 0)),
        out_shape=jax.ShapeDtypeStruct((half, 2 * D), jnp.float32),
    )


def _gather(F, V, D, B):
    total = B * F
    per_w = total // NW
    nb = per_w // F
    group = GB * F
    n_groups = nb // GB
    half_rows = F // 2 * V
    fh = F // 2
    assert total % NW == 0 and nb % GB == 0 and n_groups % 2 == 0
    assert group % 16 == 0

    mesh = plsc.VectorSubcoreMesh(core_axis_name="c", subcore_axis_name="s")

    @functools.partial(
        pl.kernel,
        out_type=jax.ShapeDtypeStruct((B, F, D), jnp.float32),
        mesh=mesh,
        compiler_params=pltpu.CompilerParams(use_tc_tiling_on_sc=True),
        scratch_types=[
            pltpu.VMEM((per_w,), jnp.int32),
            pltpu.VMEM((2, 2, 128), jnp.int32),
            pltpu.VMEM((2, GB * F, 2 * D), jnp.float32),
            pltpu.VMEM((2, GB * F, D), jnp.float32),
            pltpu.SemaphoreType.DMA,
            pltpu.SemaphoreType.DMA,
            pltpu.SemaphoreType.DMA,
            pltpu.SemaphoreType.DMA,
            pltpu.SemaphoreType.DMA,
        ],
    )
    def emb(x_hbm, pk_hbm, out_hbm, raw_v, idx_v, rows_v, out_v,
            gs0, gs1, cs, os0, os1):
        wid = lax.axis_index("s") * NC + lax.axis_index("c")
        base = wid * per_w
        b0 = wid * nb
        gsems = (gs0, gs1)
        osems = (os0, os1)

        pltpu.sync_copy(x_hbm.at[pl.ds(base, per_w)], raw_v)

        lane = lax.iota(jnp.int32, 16)

        def compute_and_fire(g, h):
            for p in range(group // 16):
                pos = g * group + p * 16
                fld = (lane + pos) % F
                r = raw_v[pl.ds(pos, 16)] + fld * V
                idx = jnp.where(fld >= fh, r - half_rows, r)
                q = p * 16
                idx_v[h, q // 128, pl.ds(q % 128, 16)] = idx
            pltpu.async_copy(
                pk_hbm.at[idx_v.at[h, 0]],
                rows_v.at[h, pl.ds(0, 128)],
                gsems[h],
            )
            pltpu.async_copy(
                pk_hbm.at[idx_v.at[h, 1, pl.ds(0, group - 128)]],
                rows_v.at[h, pl.ds(128, group - 128)],
                gsems[h],
            )

        def drain_gather(h):
            pltpu.make_async_copy(
                pk_hbm.at[idx_v.at[h, 0]],
                rows_v.at[h, pl.ds(0, 128)],
                gsems[h],
            ).wait()
            pltpu.make_async_copy(
                pk_hbm.at[idx_v.at[h, 1, pl.ds(0, group - 128)]],
                rows_v.at[h, pl.ds(128, group - 128)],
                gsems[h],
            ).wait()

        def compact_and_emit(g, h):
            def ext(j, _):
                for f in range(F):
                    off = 0 if f < fh else D
                    row = j * F + f
                    for m in range(D // 16):
                        out_v[h, row, pl.ds(m * 16, 16)] = (
                            rows_v[h, row, pl.ds(off + m * 16, 16)]
                        )
                return 0

            lax.fori_loop(0, GB, ext, 0)
            pltpu.async_copy(
                out_v.at[h].reshape(GB, F, D),
                out_hbm.at[pl.ds(b0 + g * GB, GB), :, :],
                osems[h],
            )

        def wait_ow(h):
            pltpu.make_async_copy(
                out_v.at[h].reshape(GB, F, D),
                out_hbm.at[pl.ds(b0, GB), :, :],
                osems[h],
            ).wait()

        compute_and_fire(0, 0)

        def body(sg, _):
            for h in range(2):
                g = 2 * sg + h
                nh = 1 - h

                @pl.when(g < n_groups - 1)
                def _():
                    @pl.when(g >= 1)
                    def _():
                        wait_ow(nh)
                    compute_and_fire(g + 1, nh)

                drain_gather(h)
                compact_and_emit(g, h)
            return 0

        lax.fori_loop(0, n_groups // 2, body, 0)
        wait_ow(0)
        wait_ow(1)

    return emb


def kernel(x, tables):
    F, V, D = tables.shape
    B = x.shape[0]
    xf = x.astype(jnp.int32).reshape(B * F)
    tf = tables.reshape(F * V, D)
    packed = _pack(F, V, D)(tf, tf)
    return _gather(F, V, D, B)(xf, packed)


# speedup vs baseline: 1.7323x; 1.4130x over previous
import functools

import jax
import jax.numpy as jnp
from jax import lax
from jax.experimental import pallas as pl
from jax.experimental.pallas import tpu as pltpu
from jax.experimental.pallas import tpu_sc as plsc

NC = 2
NS = 16
NW = NC * NS

GB = 8
PACK_VB = 3200


def _pack(F, V, D):
    fh = F // 2
    nvb = -(-V // PACK_VB)

    def body(a_ref, b_ref, o_ref):
        o_ref[0] = jnp.concatenate([a_ref[0].T, b_ref[0].T], axis=1)

    return pl.pallas_call(
        body,
        grid=(fh, nvb),
        in_specs=[
            pl.BlockSpec((1, D, PACK_VB), lambda i, j: (i, 0, j)),
            pl.BlockSpec((1, D, PACK_VB), lambda i, j: (i + fh, 0, j)),
        ],
        out_specs=pl.BlockSpec((1, PACK_VB, 2 * D), lambda i, j: (i, j, 0)),
        out_shape=jax.ShapeDtypeStruct((fh, V, 2 * D), jnp.float32),
    )


def _gather(F, V, D, B):
    total = B * F
    per_w = total // NW
    nb = per_w // F
    group = GB * F
    n_groups = nb // GB
    half_rows = F // 2 * V
    fh = F // 2
    assert total % NW == 0 and nb % GB == 0 and n_groups % 2 == 0
    assert group % 16 == 0

    mesh = plsc.VectorSubcoreMesh(core_axis_name="c", subcore_axis_name="s")

    @functools.partial(
        pl.kernel,
        out_type=jax.ShapeDtypeStruct((B, F, D), jnp.float32),
        mesh=mesh,
        compiler_params=pltpu.CompilerParams(use_tc_tiling_on_sc=True),
        scratch_types=[
            pltpu.VMEM((per_w,), jnp.int32),
            pltpu.VMEM((2, 2, 128), jnp.int32),
            pltpu.VMEM((2, GB * F, 2 * D), jnp.float32),
            pltpu.VMEM((2, GB * F, D), jnp.float32),
            pltpu.SemaphoreType.DMA,
            pltpu.SemaphoreType.DMA,
            pltpu.SemaphoreType.DMA,
            pltpu.SemaphoreType.DMA,
            pltpu.SemaphoreType.DMA,
        ],
    )
    def emb(x_hbm, pk_hbm, out_hbm, raw_v, idx_v, rows_v, out_v,
            gs0, gs1, cs, os0, os1):
        wid = lax.axis_index("s") * NC + lax.axis_index("c")
        base = wid * per_w
        b0 = wid * nb
        gsems = (gs0, gs1)
        osems = (os0, os1)

        pltpu.sync_copy(x_hbm.at[pl.ds(base, per_w)], raw_v)

        lane = lax.iota(jnp.int32, 16)

        def compute_and_fire(g, h):
            for p in range(group // 16):
                pos = g * group + p * 16
                fld = (lane + pos) % F
                r = raw_v[pl.ds(pos, 16)] + fld * V
                idx = jnp.where(fld >= fh, r - half_rows, r)
                q = p * 16
                idx_v[h, q // 128, pl.ds(q % 128, 16)] = idx
            pltpu.async_copy(
                pk_hbm.at[idx_v.at[h, 0]],
                rows_v.at[h, pl.ds(0, 128)],
                gsems[h],
            )
            pltpu.async_copy(
                pk_hbm.at[idx_v.at[h, 1, pl.ds(0, group - 128)]],
                rows_v.at[h, pl.ds(128, group - 128)],
                gsems[h],
            )

        def drain_gather(h):
            pltpu.make_async_copy(
                pk_hbm.at[idx_v.at[h, 0]],
                rows_v.at[h, pl.ds(0, 128)],
                gsems[h],
            ).wait()
            pltpu.make_async_copy(
                pk_hbm.at[idx_v.at[h, 1, pl.ds(0, group - 128)]],
                rows_v.at[h, pl.ds(128, group - 128)],
                gsems[h],
            ).wait()

        def compact_and_emit(g, h):
            def ext(j, _):
                for f in range(F):
                    off = 0 if f < fh else D
                    row = j * F + f
                    for m in range(D // 16):
                        out_v[h, row, pl.ds(m * 16, 16)] = (
                            rows_v[h, row, pl.ds(off + m * 16, 16)]
                        )
                return 0

            lax.fori_loop(0, GB, ext, 0)
            pltpu.async_copy(
                out_v.at[h].reshape(GB, F, D),
                out_hbm.at[pl.ds(b0 + g * GB, GB), :, :],
                osems[h],
            )

        def wait_ow(h):
            pltpu.make_async_copy(
                out_v.at[h].reshape(GB, F, D),
                out_hbm.at[pl.ds(b0, GB), :, :],
                osems[h],
            ).wait()

        compute_and_fire(0, 0)

        def body(sg, _):
            for h in range(2):
                g = 2 * sg + h
                nh = 1 - h

                @pl.when(g < n_groups - 1)
                def _():
                    @pl.when(g >= 1)
                    def _():
                        wait_ow(nh)
                    compute_and_fire(g + 1, nh)

                drain_gather(h)
                compact_and_emit(g, h)
            return 0

        lax.fori_loop(0, n_groups // 2, body, 0)
        wait_ow(0)
        wait_ow(1)

    return emb


def kernel(x, tables):
    F, V, D = tables.shape
    B = x.shape[0]
    xf = x.astype(jnp.int32).reshape(B * F)
    tt = tables.transpose(0, 2, 1)
    packed = _pack(F, V, D)(tt, tt).reshape(F // 2 * V, 2 * D)
    return _gather(F, V, D, B)(xf, packed)


# speedup vs baseline: 2.0163x; 1.1639x over previous
import functools

import jax
import jax.numpy as jnp
from jax import lax
from jax.experimental import pallas as pl
from jax.experimental.pallas import tpu as pltpu
from jax.experimental.pallas import tpu_sc as plsc

NC = 2
NS = 16
NW = NC * NS

GB = 8
PACK_VB = 3200


def _pack(F, V, D):
    fh = F // 2
    nvb = -(-V // PACK_VB)

    def body(a_ref, b_ref, o_ref):
        o_ref[0] = jnp.concatenate([a_ref[0], b_ref[0]], axis=0).T

    return pl.pallas_call(
        body,
        grid=(fh, nvb),
        in_specs=[
            pl.BlockSpec((1, D, PACK_VB), lambda i, j: (i, 0, j)),
            pl.BlockSpec((1, D, PACK_VB), lambda i, j: (i + fh, 0, j)),
        ],
        out_specs=pl.BlockSpec((1, PACK_VB, 2 * D), lambda i, j: (i, j, 0)),
        out_shape=jax.ShapeDtypeStruct((fh, V, 2 * D), jnp.float32),
    )


def _gather(F, V, D, B):
    total = B * F
    per_w = total // NW
    nb = per_w // F
    group = GB * F
    n_groups = nb // GB
    half_rows = F // 2 * V
    fh = F // 2
    assert total % NW == 0 and nb % GB == 0 and n_groups % 2 == 0
    assert group % 16 == 0

    mesh = plsc.VectorSubcoreMesh(core_axis_name="c", subcore_axis_name="s")

    @functools.partial(
        pl.kernel,
        out_type=jax.ShapeDtypeStruct((B, F, D), jnp.float32),
        mesh=mesh,
        compiler_params=pltpu.CompilerParams(use_tc_tiling_on_sc=True),
        scratch_types=[
            pltpu.VMEM((per_w,), jnp.int32),
            pltpu.VMEM((2, 2, 128), jnp.int32),
            pltpu.VMEM((2, GB * F, 2 * D), jnp.float32),
            pltpu.VMEM((2, GB * F, D), jnp.float32),
            pltpu.SemaphoreType.DMA,
            pltpu.SemaphoreType.DMA,
            pltpu.SemaphoreType.DMA,
            pltpu.SemaphoreType.DMA,
            pltpu.SemaphoreType.DMA,
        ],
    )
    def emb(x_hbm, pk_hbm, out_hbm, raw_v, idx_v, rows_v, out_v,
            gs0, gs1, cs, os0, os1):
        wid = lax.axis_index("s") * NC + lax.axis_index("c")
        base = wid * per_w
        b0 = wid * nb
        gsems = (gs0, gs1)
        osems = (os0, os1)

        pltpu.sync_copy(x_hbm.at[pl.ds(base, per_w)], raw_v)

        lane = lax.iota(jnp.int32, 16)

        def compute_and_fire(g, h):
            for p in range(group // 16):
                pos = g * group + p * 16
                fld = (lane + pos) % F
                r = raw_v[pl.ds(pos, 16)] + fld * V
                idx = jnp.where(fld >= fh, r - half_rows, r)
                q = p * 16
                idx_v[h, q // 128, pl.ds(q % 128, 16)] = idx
            pltpu.async_copy(
                pk_hbm.at[idx_v.at[h, 0]],
                rows_v.at[h, pl.ds(0, 128)],
                gsems[h],
            )
            pltpu.async_copy(
                pk_hbm.at[idx_v.at[h, 1, pl.ds(0, group - 128)]],
                rows_v.at[h, pl.ds(128, group - 128)],
                gsems[h],
            )

        def drain_gather(h):
            pltpu.make_async_copy(
                pk_hbm.at[idx_v.at[h, 0]],
                rows_v.at[h, pl.ds(0, 128)],
                gsems[h],
            ).wait()
            pltpu.make_async_copy(
                pk_hbm.at[idx_v.at[h, 1, pl.ds(0, group - 128)]],
                rows_v.at[h, pl.ds(128, group - 128)],
                gsems[h],
            ).wait()

        def compact_and_emit(g, h):
            def ext(j, _):
                for f in range(F):
                    off = 0 if f < fh else D
                    row = j * F + f
                    for m in range(D // 16):
                        out_v[h, row, pl.ds(m * 16, 16)] = (
                            rows_v[h, row, pl.ds(off + m * 16, 16)]
                        )
                return 0

            lax.fori_loop(0, GB, ext, 0)
            pltpu.async_copy(
                out_v.at[h].reshape(GB, F, D),
                out_hbm.at[pl.ds(b0 + g * GB, GB), :, :],
                osems[h],
            )

        def wait_ow(h):
            pltpu.make_async_copy(
                out_v.at[h].reshape(GB, F, D),
                out_hbm.at[pl.ds(b0, GB), :, :],
                osems[h],
            ).wait()

        compute_and_fire(0, 0)

        def body(sg, _):
            for h in range(2):
                g = 2 * sg + h
                nh = 1 - h

                @pl.when(g < n_groups - 1)
                def _():
                    @pl.when(g >= 1)
                    def _():
                        wait_ow(nh)
                    compute_and_fire(g + 1, nh)

                drain_gather(h)
                compact_and_emit(g, h)
            return 0

        lax.fori_loop(0, n_groups // 2, body, 0)
        wait_ow(0)
        wait_ow(1)

    return emb


def kernel(x, tables):
    F, V, D = tables.shape
    B = x.shape[0]
    xf = x.astype(jnp.int32).reshape(B * F)
    tt = tables.transpose(0, 2, 1)
    packed = _pack(F, V, D)(tt, tt).reshape(F // 2 * V, 2 * D)
    return _gather(F, V, D, B)(xf, packed)
